# baseline (speedup 1.0000x reference)
import functools

import jax
import jax.numpy as jnp
from jax import lax
from jax.experimental import pallas as pl
from jax.experimental.pallas import tpu as pltpu
from jax.experimental.pallas import tpu_sc as plsc

D_MODEL = 64
NUM_CORES = 2
NUM_SUBCORES = 16
NUM_WORKERS = NUM_CORES * NUM_SUBCORES


@functools.partial(jax.jit, static_argnums=(2, 3))
def _gather_rows(table, idx, b_per_w, chunk):
    nchunks = b_per_w // chunk
    total = b_per_w * NUM_WORKERS
    mesh = plsc.VectorSubcoreMesh(core_axis_name="c", subcore_axis_name="s")

    @functools.partial(
        pl.kernel,
        out_type=jax.ShapeDtypeStruct((total, D_MODEL), jnp.float32),
        mesh=mesh,
        scratch_types=[
            pltpu.VMEM((b_per_w,), jnp.int32),
            pltpu.VMEM((chunk, D_MODEL), jnp.float32),
            pltpu.SemaphoreType.DMA,
        ],
        compiler_params=pltpu.CompilerParams(use_tc_tiling_on_sc=False),
    )
    def k(table_hbm, idx_hbm, out_hbm, idx_v, rows_v, sem):
        wid = lax.axis_index("s") * NUM_CORES + lax.axis_index("c")
        base = wid * b_per_w
        pltpu.sync_copy(idx_hbm.at[pl.ds(base, b_per_w)], idx_v)

        def body(g, carry):
            off = g * chunk
            pltpu.async_copy(
                table_hbm.at[idx_v.at[pl.ds(off, chunk)]], rows_v, sem
            ).wait()
            pltpu.sync_copy(rows_v, out_hbm.at[pl.ds(base + off, chunk)])
            return carry

        lax.fori_loop(0, nchunks, body, 0)

    return k(table, idx)


def kernel(x, table):
    flat = x.reshape(-1).astype(jnp.int32)
    b_per_w = flat.shape[0] // NUM_WORKERS
    out = _gather_rows(table, flat, b_per_w, 128)
    return out.reshape(x.shape + (table.shape[1],))


# speedup vs baseline: 1.0456x; 1.0456x over previous
import functools

import jax
import jax.numpy as jnp
from jax import lax
from jax.experimental import pallas as pl
from jax.experimental.pallas import tpu as pltpu
from jax.experimental.pallas import tpu_sc as plsc

D_MODEL = 64
NUM_CORES = 2
NUM_SUBCORES = 16
NUM_WORKERS = NUM_CORES * NUM_SUBCORES
CHUNK = 128


@functools.partial(jax.jit, static_argnums=(2, 3))
def _gather_rows(table, idx, b_per_w, group):
    k = group // CHUNK
    ngroups = b_per_w // group
    assert ngroups % 2 == 0 and ngroups >= 4
    total = b_per_w * NUM_WORKERS
    mesh = plsc.VectorSubcoreMesh(core_axis_name="c", subcore_axis_name="s")

    @functools.partial(
        pl.kernel,
        out_type=jax.ShapeDtypeStruct((total, D_MODEL), jnp.float32),
        mesh=mesh,
        scratch_types=[
            pltpu.VMEM((b_per_w,), jnp.int32),
            pltpu.VMEM((group, D_MODEL), jnp.float32),
            pltpu.VMEM((group, D_MODEL), jnp.float32),
            pltpu.SemaphoreType.DMA,
            pltpu.SemaphoreType.DMA,
            pltpu.SemaphoreType.DMA,
            pltpu.SemaphoreType.DMA,
        ],
        compiler_params=pltpu.CompilerParams(use_tc_tiling_on_sc=False),
    )
    def kern(table_hbm, idx_hbm, out_hbm, idx_v, rows_a, rows_b,
             gsem_a, gsem_b, wsem_a, wsem_b):
        wid = lax.axis_index("s") * NUM_CORES + lax.axis_index("c")
        base = wid * b_per_w
        pltpu.sync_copy(idx_hbm.at[pl.ds(base, b_per_w)], idx_v)

        def gathers_start(g, buf, sem):
            for i in range(k):
                off = g * group + i * CHUNK
                pltpu.async_copy(
                    table_hbm.at[idx_v.at[pl.ds(off, CHUNK)]],
                    buf.at[pl.ds(i * CHUNK, CHUNK)], sem)

        def gathers_wait(buf, sem):
            for i in range(k):
                pltpu.make_async_copy(
                    table_hbm.at[idx_v.at[pl.ds(0, CHUNK)]],
                    buf.at[pl.ds(i * CHUNK, CHUNK)], sem).wait()

        def write_start(g, buf, sem):
            pltpu.async_copy(buf, out_hbm.at[pl.ds(base + g * group, group)],
                             sem)

        def write_wait(buf, sem):
            pltpu.make_async_copy(buf, out_hbm.at[pl.ds(base, group)],
                                  sem).wait()

        gathers_start(0, rows_a, gsem_a)
        gathers_start(1, rows_b, gsem_b)
        gathers_wait(rows_a, gsem_a)
        write_start(0, rows_a, wsem_a)

        def body(p, carry):
            ga = 2 * p + 2
            write_wait(rows_a, wsem_a)
            gathers_start(ga, rows_a, gsem_a)
            gathers_wait(rows_b, gsem_b)
            write_start(ga - 1, rows_b, wsem_b)
            write_wait(rows_b, wsem_b)
            gathers_start(ga + 1, rows_b, gsem_b)
            gathers_wait(rows_a, gsem_a)
            write_start(ga, rows_a, wsem_a)
            return carry

        lax.fori_loop(0, (ngroups - 2) // 2, body, 0)

        gathers_wait(rows_b, gsem_b)
        write_start(ngroups - 1, rows_b, wsem_b)
        write_wait(rows_a, wsem_a)
        write_wait(rows_b, wsem_b)

    return kern(table, idx)


def kernel(x, table):
    flat = x.reshape(-1).astype(jnp.int32)
    b_per_w = flat.shape[0] // NUM_WORKERS
    out = _gather_rows(table, flat, b_per_w, 640)
    return out.reshape(x.shape + (table.shape[1],))


# speedup vs baseline: 1.0493x; 1.0035x over previous
import functools

import jax
import jax.numpy as jnp
from jax import lax
from jax.experimental import pallas as pl
from jax.experimental.pallas import tpu as pltpu
from jax.experimental.pallas import tpu_sc as plsc

D_MODEL = 64
NUM_CORES = 2
NUM_SUBCORES = 16
NUM_WORKERS = NUM_CORES * NUM_SUBCORES
CHUNK = 128
GROUP = 5


@jax.jit
def _embed(table, xt):
    seq, batch = xt.shape
    ngroups = seq // GROUP
    assert ngroups % 2 == 0 and ngroups >= 4
    assert batch == CHUNK * NUM_WORKERS
    mesh = plsc.VectorSubcoreMesh(core_axis_name="c", subcore_axis_name="s")

    @functools.partial(
        pl.kernel,
        out_type=jax.ShapeDtypeStruct((batch, seq, D_MODEL), jnp.float32),
        mesh=mesh,
        scratch_types=[
            pltpu.VMEM((seq, CHUNK), jnp.int32),
            pltpu.VMEM((GROUP * CHUNK, D_MODEL), jnp.float32),
            pltpu.VMEM((GROUP * CHUNK, D_MODEL), jnp.float32),
            pltpu.SemaphoreType.DMA,
            pltpu.SemaphoreType.DMA,
            pltpu.SemaphoreType.DMA,
            pltpu.SemaphoreType.DMA,
        ],
        compiler_params=pltpu.CompilerParams(use_tc_tiling_on_sc=False),
    )
    def kern(table_hbm, xt_hbm, out_hbm, idx_v, rows_a, rows_b,
             gsem_a, gsem_b, wsem_a, wsem_b):
        wid = lax.axis_index("s") * NUM_CORES + lax.axis_index("c")
        base_b = wid * CHUNK
        pltpu.sync_copy(xt_hbm.at[:, pl.ds(base_b, CHUNK)], idx_v)

        def gathers_start(g, buf, sem):
            for i in range(GROUP):
                pltpu.async_copy(
                    table_hbm.at[idx_v.at[g * GROUP + i]],
                    buf.at[pl.ds(i * CHUNK, CHUNK)], sem)

        def gathers_wait(buf, sem):
            for i in range(GROUP):
                pltpu.make_async_copy(
                    table_hbm.at[idx_v.at[0]],
                    buf.at[pl.ds(i * CHUNK, CHUNK)], sem).wait()

        def writes_start(g, buf, sem):
            for i in range(GROUP):
                pltpu.async_copy(
                    buf.at[pl.ds(i * CHUNK, CHUNK)],
                    out_hbm.at[pl.ds(base_b, CHUNK), g * GROUP + i], sem)

        def writes_wait(buf, sem):
            for i in range(GROUP):
                pltpu.make_async_copy(
                    buf.at[pl.ds(i * CHUNK, CHUNK)],
                    out_hbm.at[pl.ds(base_b, CHUNK), 0], sem).wait()

        gathers_start(0, rows_a, gsem_a)
        gathers_start(1, rows_b, gsem_b)
        gathers_wait(rows_a, gsem_a)
        writes_start(0, rows_a, wsem_a)

        def body(p, carry):
            ga = 2 * p + 2
            writes_wait(rows_a, wsem_a)
            gathers_start(ga, rows_a, gsem_a)
            gathers_wait(rows_b, gsem_b)
            writes_start(ga - 1, rows_b, wsem_b)
            writes_wait(rows_b, wsem_b)
            gathers_start(ga + 1, rows_b, gsem_b)
            gathers_wait(rows_a, gsem_a)
            writes_start(ga, rows_a, wsem_a)
            return carry

        lax.fori_loop(0, (ngroups - 2) // 2, body, 0)

        gathers_wait(rows_b, gsem_b)
        writes_start(ngroups - 1, rows_b, wsem_b)
        writes_wait(rows_a, wsem_a)
        writes_wait(rows_b, wsem_b)

    return kern(table, xt)


def kernel(x, table):
    xt = jnp.swapaxes(x, 0, 1).astype(jnp.int32)
    return _embed(table, xt)
